# baseline (speedup 1.0000x reference)
import jax
import jax.numpy as jnp
from jax.experimental import pallas as pl
from jax.experimental.pallas import tpu as pltpu

S = 128
P = 16
H = 64
E = 64
MID = 128
BOT = 1024
EPS = 1e-5
G = 64


def _body(pos_ref, h_ref, wse_ref, w1_ref, b1_ref, g1_ref, be1_ref,
          w2_ref, b2_ref, g2_ref, be2_ref, out_ref, w2f_ref):
    inv = 1.0 / jnp.sqrt(1.0 + EPS)

    @pl.when(pl.program_id(0) == 0)
    def _fold_w2():
        a2 = g2_ref[...] * inv
        w2f_ref[...] = (w2_ref[...] * a2).astype(jnp.bfloat16)

    pos = pos_ref[...]
    h = h_ref[...]
    wse = wse_ref[...]

    q = pos[:, 0:1] * wse[0:1, :] + pos[:, 1:2] * wse[1:2, :]

    w1 = w1_ref[...]
    r = jnp.dot(q, w1[:E, :], preferred_element_type=jnp.float32)
    t = jnp.dot(h, w1[E:, :], preferred_element_type=jnp.float32)

    a1 = g1_ref[...] * inv
    rf = a1 * r
    uf = a1 * (r + t + b1_ref[...]) + be1_ref[...]

    uft = jnp.transpose(uf.reshape(G, P, MID), (1, 0, 2))
    rf3 = rf.reshape(G, P, MID)
    w2f = w2f_ref[...]
    m = None
    for j in range(0, P, 2):
        y0 = jnp.maximum(uft[j].reshape(G, 1, MID) - rf3, 0.0)
        y1 = jnp.maximum(uft[j + 1].reshape(G, 1, MID) - rf3, 0.0)
        z0 = jnp.dot(y0.reshape(G * P, MID).astype(jnp.bfloat16), w2f,
                     preferred_element_type=jnp.float32)
        z1 = jnp.dot(y1.reshape(G * P, MID).astype(jnp.bfloat16), w2f,
                     preferred_element_type=jnp.float32)
        zp = jnp.maximum(z0, z1)
        m = zp if m is None else jnp.maximum(m, zp)

    a2 = g2_ref[...] * inv
    c2 = a2 * b2_ref[...] + be2_ref[...]
    out_ref[...] = jnp.maximum(m + c2, 0.0)


@jax.jit
def kernel(h_states, seq_start_end, end_pos, W_se, b_se, W1, b1, g1, be1,
           W2, b2, g2, be2):
    del seq_start_end, b_se
    h = h_states.reshape(S * P, H)
    pos = end_pos

    full = lambda shape: pl.BlockSpec(shape, lambda i: (0,) * len(shape))
    out = pl.pallas_call(
        _body,
        grid=(S // G,),
        in_specs=[
            pl.BlockSpec((G * P, 2), lambda i: (i, 0)),
            pl.BlockSpec((G * P, H), lambda i: (i, 0)),
            full((2, E)),
            full((E + H, MID)),
            full((MID,)),
            full((MID,)),
            full((MID,)),
            full((MID, BOT)),
            full((BOT,)),
            full((BOT,)),
            full((BOT,)),
        ],
        out_specs=pl.BlockSpec((G * P, BOT), lambda i: (i, 0)),
        out_shape=jax.ShapeDtypeStruct((S * P, BOT), jnp.float32),
        scratch_shapes=[pltpu.VMEM((MID, BOT), jnp.bfloat16)],
    )(pos, h, W_se, W1, b1, g1,
      be1, W2, b2, g2, be2)
    return out


# speedup vs baseline: 1.2130x; 1.2130x over previous
import jax
import jax.numpy as jnp
from jax.experimental import pallas as pl
from jax.experimental.pallas import tpu as pltpu

S = 128
P = 16
H = 64
E = 64
MID = 128
BOT = 1024
EPS = 1e-5
G = 64


def _body(pos_ref, h_ref, wse_ref, w1_ref, b1_ref, g1_ref, be1_ref,
          w2_ref, b2_ref, g2_ref, be2_ref, out_ref, w2f_ref):
    inv = 1.0 / jnp.sqrt(1.0 + EPS)

    @pl.when(pl.program_id(0) == 0)
    def _fold_w2():
        a2 = g2_ref[...] * inv
        w2f_ref[...] = (w2_ref[...] * a2).astype(jnp.bfloat16)

    post = pos_ref[...]
    ht = h_ref[...]
    wse = wse_ref[...]

    w1 = w1_ref[...]
    wse_w1a = jnp.dot(wse, w1[:E, :], preferred_element_type=jnp.float32)
    dn = (((0,), (0,)), ((), ()))
    r = jax.lax.dot_general(post, wse_w1a, dn,
                            preferred_element_type=jnp.float32)
    t = jax.lax.dot_general(ht, w1[E:, :], dn,
                            preferred_element_type=jnp.float32)

    a1 = g1_ref[...] * inv
    rf = a1 * r
    uf = a1 * (r + t + b1_ref[...]) + be1_ref[...]

    uft = jnp.transpose(uf.reshape(G, P, MID), (1, 0, 2))
    rf3 = rf.reshape(G, P, MID)
    w2f = w2f_ref[...]
    m = None
    for j in range(0, P, 2):
        y0 = jnp.maximum(uft[j].reshape(G, 1, MID) - rf3, 0.0)
        y1 = jnp.maximum(uft[j + 1].reshape(G, 1, MID) - rf3, 0.0)
        z0 = jnp.dot(y0.reshape(G * P, MID).astype(jnp.bfloat16), w2f,
                     preferred_element_type=jnp.float32)
        z1 = jnp.dot(y1.reshape(G * P, MID).astype(jnp.bfloat16), w2f,
                     preferred_element_type=jnp.float32)
        zp = jnp.maximum(z0, z1)
        m = zp if m is None else jnp.maximum(m, zp)

    a2 = g2_ref[...] * inv
    c2 = a2 * b2_ref[...] + be2_ref[...]
    out_ref[...] = jnp.maximum(m + c2, 0.0)


@jax.jit
def kernel(h_states, seq_start_end, end_pos, W_se, b_se, W1, b1, g1, be1,
           W2, b2, g2, be2):
    del seq_start_end, b_se
    ht = h_states.reshape(S * P, H).T
    post = end_pos.T

    full = lambda shape: pl.BlockSpec(shape, lambda i: (0,) * len(shape))
    out = pl.pallas_call(
        _body,
        grid=(S // G,),
        in_specs=[
            pl.BlockSpec((2, G * P), lambda i: (0, i)),
            pl.BlockSpec((H, G * P), lambda i: (0, i)),
            full((2, E)),
            full((E + H, MID)),
            full((MID,)),
            full((MID,)),
            full((MID,)),
            full((MID, BOT)),
            full((BOT,)),
            full((BOT,)),
            full((BOT,)),
        ],
        out_specs=pl.BlockSpec((G * P, BOT), lambda i: (i, 0)),
        out_shape=jax.ShapeDtypeStruct((S * P, BOT), jnp.float32),
        scratch_shapes=[pltpu.VMEM((MID, BOT), jnp.bfloat16)],
    )(post, ht, W_se, W1, b1, g1,
      be1, W2, b2, g2, be2)
    return out
